# baseline (speedup 1.0000x reference)
import jax
import jax.numpy as jnp
from jax import lax
from jax.experimental import pallas as pl
from jax.experimental.pallas import tpu as pltpu
from jax.experimental.pallas import tpu_sc as plsc

B = 16384
HIST = 50
D = 64
VOCAB = 1000000

NC = 2
NS = 16
NW = NC * NS

U_PER_W = B // NW
CHUNK_U = 8
N_CHUNKS = U_PER_W // CHUNK_U
ROWS = CHUNK_U * HIST
IDX_MINOR = 80
IDX_MAJOR = ROWS // IDX_MINOR


def _sc_body(uid_hbm, midx_hbm, utab2_hbm, mtab2_hbm, out_hbm, *scratch):
    bufs = (scratch[0:7], scratch[7:14])
    sems = scratch[14:16]
    wid = lax.axis_index("s") * NC + lax.axis_index("c")
    wbase = wid * U_PER_W

    lane = lax.iota(jnp.int32, 16)
    last_lane = lane == 15
    bfly = [lane ^ d for d in (8, 4, 2, 1)]

    for b in range(2):
        bufs[b][2][pl.ds(0, 16)] = jnp.zeros((16,), jnp.int32)
        bufs[b][2][pl.ds(16, 16)] = jnp.zeros((16,), jnp.int32)

    def issue(c, b):
        midx_v, midx2_v, uidx_v, uidx2_v, mrows_v, urows_v, _ = bufs[b]
        sem = sems[b]
        ubase = wbase + c * CHUNK_U
        pltpu.sync_copy(midx_hbm.at[pl.ds(ubase * HIST, ROWS)],
                        midx_v.at[pl.ds(0, ROWS)])
        pltpu.sync_copy(uid_hbm.at[pl.ds(ubase, CHUNK_U)],
                        uidx_v.at[pl.ds(0, CHUNK_U)])
        for k in range(ROWS // 16):
            midx2_v[pl.ds(k * 16, 16)] = lax.shift_right_logical(
                midx_v[pl.ds(k * 16, 16)], 1)
        uidx2_v[...] = lax.shift_right_logical(uidx_v[pl.ds(0, 16)], 1)
        for j in range(IDX_MAJOR):
            pltpu.async_copy(
                mtab2_hbm.at[midx2_v.at[pl.ds(j * IDX_MINOR, IDX_MINOR)]],
                mrows_v.at[pl.ds(j * IDX_MINOR, IDX_MINOR)], sem)
        pltpu.async_copy(utab2_hbm.at[uidx2_v], urows_v, sem)

    def wait(b):
        midx_v, midx2_v, uidx_v, uidx2_v, mrows_v, urows_v, _ = bufs[b]
        sem = sems[b]
        for j in range(IDX_MAJOR):
            pltpu.make_async_copy(
                mtab2_hbm.at[midx2_v.at[pl.ds(j * IDX_MINOR, IDX_MINOR)]],
                mrows_v.at[pl.ds(j * IDX_MINOR, IDX_MINOR)], sem).wait()
        pltpu.make_async_copy(utab2_hbm.at[uidx2_v], urows_v, sem).wait()

    def compute(c, b):
        midx_v, _, uidx_v, _, mrows_v, urows_v, out_v = bufs[b]
        ubase = wbase + c * CHUNK_U

        def user_body(i, _):
            uhb = (uidx_v[pl.ds(i, 16)][0] & 1) * D
            u0 = urows_v[i, pl.ds(uhb, 16)]
            u1 = urows_v[i, pl.ds(uhb + 16, 16)]
            u2 = urows_v[i, pl.ds(uhb + 32, 16)]
            u3 = urows_v[i, pl.ds(uhb + 48, 16)]
            base_row = i * HIST
            for h in range(HIST):
                r = base_row + h
                mhb = (midx_v[pl.ds(r, 16)][0] & 1) * D
                p = (mrows_v[r, pl.ds(mhb, 16)] * u0
                     + mrows_v[r, pl.ds(mhb + 16, 16)] * u1
                     + mrows_v[r, pl.ds(mhb + 32, 16)] * u2
                     + mrows_v[r, pl.ds(mhb + 48, 16)] * u3)
                for ix in bfly:
                    p = p + p.at[ix].get(mode="promise_in_bounds")
                plsc.store_scatter(out_v, [jnp.full((16,), r, jnp.int32)],
                                   p, mask=last_lane)
            return 0

        lax.fori_loop(0, CHUNK_U, user_body, 0)
        pltpu.sync_copy(out_v, out_hbm.at[pl.ds(ubase * HIST, ROWS)])

    issue(0, 0)

    def pair_body(gp, _):
        c0 = gp * 2
        issue(c0 + 1, 1)
        wait(0)
        compute(c0, 0)

        @pl.when(c0 + 2 < N_CHUNKS)
        def _():
            issue(c0 + 2, 0)

        wait(1)
        compute(c0 + 1, 1)
        return 0

    lax.fori_loop(0, N_CHUNKS // 2, pair_body, 0)


@jax.jit
def _run(uid_flat, midx_flat, utab2, mtab2):
    mesh = plsc.VectorSubcoreMesh(core_axis_name="c", subcore_axis_name="s")
    buf_types = [
        pltpu.VMEM((ROWS + 16,), jnp.int32),
        pltpu.VMEM((ROWS,), jnp.int32),
        pltpu.VMEM((32,), jnp.int32),
        pltpu.VMEM((16,), jnp.int32),
        pltpu.VMEM((ROWS, 2 * D), jnp.float32),
        pltpu.VMEM((16, 2 * D), jnp.float32),
        pltpu.VMEM((ROWS,), jnp.float32),
    ]
    k = pl.kernel(
        _sc_body,
        out_type=jax.ShapeDtypeStruct((B * HIST,), jnp.float32),
        mesh=mesh,
        scratch_types=buf_types + buf_types + [pltpu.SemaphoreType.DMA,
                                               pltpu.SemaphoreType.DMA],
        compiler_params=pltpu.CompilerParams(needs_layout_passes=False),
    )
    return k(uid_flat, midx_flat, utab2, mtab2)


def kernel(user_id, movie_title, user_table, movie_table):
    uid_flat = user_id.reshape(B)
    midx_flat = movie_title.reshape(B * HIST)
    utab2 = user_table.reshape(VOCAB // 2, 2 * D)
    mtab2 = movie_table.reshape(VOCAB // 2, 2 * D)
    out = _run(uid_flat, midx_flat, utab2, mtab2)
    return out.reshape(B, HIST)


# speedup vs baseline: 1.4276x; 1.4276x over previous
import jax
import jax.numpy as jnp
from jax import lax
from jax.experimental import pallas as pl
from jax.experimental.pallas import tpu as pltpu
from jax.experimental.pallas import tpu_sc as plsc

B = 16384
HIST = 50
D = 64

NC = 2
NS = 16
NW = NC * NS

U_PER_W = B // NW
CHUNK_U = 16
N_CHUNKS = U_PER_W // CHUNK_U
ROWS = CHUNK_U * HIST
IDX_MINOR = 80
IDX_MAJOR = ROWS // IDX_MINOR


def _sc_body(uid_hbm, midx_hbm, utab_hbm, mtab_hbm, out_hbm, *scratch):
    bufs = (scratch[0:4], scratch[4:8])
    sems = scratch[8:10]
    out_v = scratch[10]
    wid = lax.axis_index("s") * NC + lax.axis_index("c")
    wbase = wid * U_PER_W

    lane = lax.iota(jnp.int32, 16)
    out_mask = (lane == 7) | (lane == 15)
    lo_half = lane < 8
    out_off = jnp.where(lo_half, 0, 1)
    bfly = [lane ^ d for d in (8, 4, 2, 1)]

    def issue(c, b):
        midx_v, uidx_v, mrows_v, urows_v = bufs[b]
        sem = sems[b]
        ubase = wbase + c * CHUNK_U
        pltpu.sync_copy(midx_hbm.at[pl.ds(ubase * HIST, ROWS)], midx_v)
        pltpu.sync_copy(uid_hbm.at[pl.ds(ubase, CHUNK_U)], uidx_v)
        for j in range(IDX_MAJOR):
            pltpu.async_copy(
                mtab_hbm.at[midx_v.at[pl.ds(j * IDX_MINOR, IDX_MINOR)]],
                mrows_v.at[pl.ds(j * IDX_MINOR, IDX_MINOR)], sem)
        pltpu.async_copy(utab_hbm.at[uidx_v], urows_v, sem)

    def wait(b):
        midx_v, uidx_v, mrows_v, urows_v = bufs[b]
        sem = sems[b]
        for j in range(IDX_MAJOR):
            pltpu.make_async_copy(
                mtab_hbm.at[midx_v.at[pl.ds(j * IDX_MINOR, IDX_MINOR)]],
                mrows_v.at[pl.ds(j * IDX_MINOR, IDX_MINOR)], sem).wait()
        pltpu.make_async_copy(utab_hbm.at[uidx_v], urows_v, sem).wait()

    def compute(c, b):
        mrows_v, urows_v = bufs[b][2], bufs[b][3]
        ubase = wbase + c * CHUNK_U

        def user_body(i, _):
            u0 = urows_v[i, pl.ds(0, 16)]
            u1 = urows_v[i, pl.ds(16, 16)]
            u2 = urows_v[i, pl.ds(32, 16)]
            u3 = urows_v[i, pl.ds(48, 16)]
            base_row = i * HIST

            def dot(r):
                return (mrows_v[r, pl.ds(0, 16)] * u0
                        + mrows_v[r, pl.ds(16, 16)] * u1
                        + mrows_v[r, pl.ds(32, 16)] * u2
                        + mrows_v[r, pl.ds(48, 16)] * u3)

            for h in range(0, HIST, 2):
                ra = base_row + h
                pa = dot(ra)
                pb = dot(ra + 1)
                sa = pa + pa.at[bfly[0]].get(mode="promise_in_bounds")
                sb = pb + pb.at[bfly[0]].get(mode="promise_in_bounds")
                s = jnp.where(lo_half, sa, sb)
                for ix in bfly[1:]:
                    s = s + s.at[ix].get(mode="promise_in_bounds")
                plsc.store_scatter(out_v, [out_off + ra], s, mask=out_mask)
            return 0

        lax.fori_loop(0, CHUNK_U, user_body, 0)
        pltpu.sync_copy(out_v, out_hbm.at[pl.ds(ubase * HIST, ROWS)])

    issue(0, 0)

    def pair_body(gp, _):
        c0 = gp * 2
        issue(c0 + 1, 1)
        wait(0)
        compute(c0, 0)

        @pl.when(c0 + 2 < N_CHUNKS)
        def _():
            issue(c0 + 2, 0)

        wait(1)
        compute(c0 + 1, 1)
        return 0

    lax.fori_loop(0, N_CHUNKS // 2, pair_body, 0)


@jax.jit
def _run(uid_flat, midx_flat, user_table, movie_table):
    mesh = plsc.VectorSubcoreMesh(core_axis_name="c", subcore_axis_name="s")
    buf_types = [
        pltpu.VMEM((ROWS,), jnp.int32),
        pltpu.VMEM((CHUNK_U,), jnp.int32),
        pltpu.VMEM((ROWS, D), jnp.float32),
        pltpu.VMEM((CHUNK_U, D), jnp.float32),
    ]
    k = pl.kernel(
        _sc_body,
        out_type=jax.ShapeDtypeStruct((B * HIST,), jnp.float32),
        mesh=mesh,
        scratch_types=buf_types + buf_types + [
            pltpu.SemaphoreType.DMA,
            pltpu.SemaphoreType.DMA,
            pltpu.VMEM((ROWS,), jnp.float32),
        ],
        compiler_params=pltpu.CompilerParams(needs_layout_passes=False,
                                             use_tc_tiling_on_sc=False),
    )
    return k(uid_flat, midx_flat, user_table, movie_table)


def kernel(user_id, movie_title, user_table, movie_table):
    uid_flat = user_id.reshape(B)
    midx_flat = movie_title.reshape(B * HIST)
    out = _run(uid_flat, midx_flat, user_table, movie_table)
    return out.reshape(B, HIST)


# speedup vs baseline: 1.4644x; 1.0258x over previous
import jax
import jax.numpy as jnp
from jax import lax
from jax.experimental import pallas as pl
from jax.experimental.pallas import tpu as pltpu
from jax.experimental.pallas import tpu_sc as plsc

B = 16384
HIST = 50
D = 64

NC = 2
NS = 16
NW = NC * NS

U_PER_W = B // NW
CHUNK_U = 8
N_CHUNKS = U_PER_W // CHUNK_U
ROWS = CHUNK_U * HIST
IDX_SPLIT = (128, 128, 128, 16)
U_SPLIT = (128, 128, 128, 128)


def _sc_body(uid_hbm, midx_hbm, utab_hbm, mtab_hbm, out_hbm,
             midx_v, uidx_v, urows_v,
             mrows0_v, mrows1_v, out0_v, out1_v,
             gsem0, gsem1, usem, osem0, osem1):
    mrows = (mrows0_v, mrows1_v)
    outs = (out0_v, out1_v)
    gsems = (gsem0, gsem1)
    osems = (osem0, osem1)
    wid = lax.axis_index("s") * NC + lax.axis_index("c")
    wbase = wid * U_PER_W

    lane = lax.iota(jnp.int32, 16)
    out_mask = (lane == 7) | (lane == 15)
    lo_half = lane < 8
    out_off = jnp.where(lo_half, 0, 1)
    bfly = [lane ^ d for d in (8, 4, 2, 1)]

    def movie_descs(c, b):
        descs = []
        off = 0
        for n in IDX_SPLIT:
            descs.append(pltpu.make_async_copy(
                mtab_hbm.at[midx_v.at[pl.ds(c * ROWS + off, n)]],
                mrows[b].at[pl.ds(off, n)], gsems[b]))
            off += n
        return descs

    def issue(c, b):
        for d in movie_descs(c, b):
            d.start()

    def wait(b):
        for d in movie_descs(0, b):
            d.wait()

    def out_desc(c, b):
        return pltpu.make_async_copy(
            outs[b], out_hbm.at[pl.ds((wbase + c * CHUNK_U) * HIST, ROWS)],
            osems[b])

    def compute(c, b):
        mrows_v = mrows[b]
        out_v = outs[b]

        @pl.when(c >= 2)
        def _():
            out_desc(c - 2, b).wait()

        def user_body(i, _):
            g = c * CHUNK_U + i
            u0 = urows_v[g, pl.ds(0, 16)]
            u1 = urows_v[g, pl.ds(16, 16)]
            u2 = urows_v[g, pl.ds(32, 16)]
            u3 = urows_v[g, pl.ds(48, 16)]
            base_row = i * HIST

            def dot(r):
                return (mrows_v[r, pl.ds(0, 16)] * u0
                        + mrows_v[r, pl.ds(16, 16)] * u1
                        + mrows_v[r, pl.ds(32, 16)] * u2
                        + mrows_v[r, pl.ds(48, 16)] * u3)

            for h in range(0, HIST, 2):
                ra = base_row + h
                pa = dot(ra)
                pb = dot(ra + 1)
                sa = pa + pa.at[bfly[0]].get(mode="promise_in_bounds")
                sb = pb + pb.at[bfly[0]].get(mode="promise_in_bounds")
                s = jnp.where(lo_half, sa, sb)
                for ix in bfly[1:]:
                    s = s + s.at[ix].get(mode="promise_in_bounds")
                plsc.store_scatter(out_v, [out_off + ra], s, mask=out_mask)
            return 0

        lax.fori_loop(0, CHUNK_U, user_body, 0)
        out_desc(c, b).start()

    pltpu.sync_copy(midx_hbm.at[pl.ds(wbase * HIST, U_PER_W * HIST)], midx_v)
    pltpu.sync_copy(uid_hbm.at[pl.ds(wbase, U_PER_W)], uidx_v)
    udescs = []
    uoff = 0
    for n in U_SPLIT:
        udescs.append(pltpu.make_async_copy(
            utab_hbm.at[uidx_v.at[pl.ds(uoff, n)]],
            urows_v.at[pl.ds(uoff, n)], usem))
        uoff += n
    for d in udescs:
        d.start()
    issue(0, 0)
    for d in udescs:
        d.wait()

    def pair_body(gp, _):
        c0 = gp * 2
        issue(c0 + 1, 1)
        wait(0)
        compute(c0, 0)

        @pl.when(c0 + 2 < N_CHUNKS)
        def _():
            issue(c0 + 2, 0)

        wait(1)
        compute(c0 + 1, 1)
        return 0

    lax.fori_loop(0, N_CHUNKS // 2, pair_body, 0)

    out_desc(N_CHUNKS - 2, 0).wait()
    out_desc(N_CHUNKS - 1, 1).wait()


@jax.jit
def _run(uid_flat, midx_flat, user_table, movie_table):
    mesh = plsc.VectorSubcoreMesh(core_axis_name="c", subcore_axis_name="s")
    k = pl.kernel(
        _sc_body,
        out_type=jax.ShapeDtypeStruct((B * HIST,), jnp.float32),
        mesh=mesh,
        scratch_types=[
            pltpu.VMEM((U_PER_W * HIST,), jnp.int32),
            pltpu.VMEM((U_PER_W,), jnp.int32),
            pltpu.VMEM((U_PER_W, D), jnp.float32),
            pltpu.VMEM((ROWS, D), jnp.float32),
            pltpu.VMEM((ROWS, D), jnp.float32),
            pltpu.VMEM((ROWS,), jnp.float32),
            pltpu.VMEM((ROWS,), jnp.float32),
            pltpu.SemaphoreType.DMA,
            pltpu.SemaphoreType.DMA,
            pltpu.SemaphoreType.DMA,
            pltpu.SemaphoreType.DMA,
            pltpu.SemaphoreType.DMA,
        ],
        compiler_params=pltpu.CompilerParams(needs_layout_passes=False,
                                             use_tc_tiling_on_sc=False),
    )
    return k(uid_flat, midx_flat, user_table, movie_table)


def kernel(user_id, movie_title, user_table, movie_table):
    uid_flat = user_id.reshape(B)
    midx_flat = movie_title.reshape(B * HIST)
    out = _run(uid_flat, midx_flat, user_table, movie_table)
    return out.reshape(B, HIST)


# speedup vs baseline: 1.7312x; 1.1822x over previous
import jax
import jax.numpy as jnp
from jax import lax
from jax.experimental import pallas as pl
from jax.experimental.pallas import tpu as pltpu
from jax.experimental.pallas import tpu_sc as plsc

B = 16384
HIST = 50
D = 64

NC = 2
NS = 16
NW = NC * NS

U_PER_W = B // NW
CHUNK_U = 8
N_CHUNKS = U_PER_W // CHUNK_U
ROWS = CHUNK_U * HIST
IDX_SPLIT = (128, 128, 128, 16)
URING = 8


def _sc_body(urows_hbm, midx_hbm, mtab_hbm, out_hbm,
             midx_v, urows_v,
             mrows0_v, mrows1_v, out0_v, out1_v,
             gsem0, gsem1, osem0, osem1):
    mrows = (mrows0_v, mrows1_v)
    outs = (out0_v, out1_v)
    gsems = (gsem0, gsem1)
    osems = (osem0, osem1)
    wid = lax.axis_index("s") * NC + lax.axis_index("c")
    wbase = wid * U_PER_W

    lane = lax.iota(jnp.int32, 16)
    out_mask = (lane == 7) | (lane == 15)
    lo_half = lane < 8
    out_off = jnp.where(lo_half, 0, 1)
    bfly = [lane ^ d for d in (8, 4, 2, 1)]

    def movie_descs(c, b):
        descs = []
        off = 0
        for n in IDX_SPLIT:
            descs.append(pltpu.make_async_copy(
                mtab_hbm.at[midx_v.at[pl.ds(c * ROWS + off, n)]],
                mrows[b].at[pl.ds(off, n)], gsems[b]))
            off += n
        return descs

    def issue(c, b):
        for d in movie_descs(c, b):
            d.start()

    def wait(b):
        for d in movie_descs(0, b):
            d.wait()

    def out_desc(c, b):
        return pltpu.make_async_copy(
            outs[b], out_hbm.at[pl.ds((wbase + c * CHUNK_U) * HIST, ROWS)],
            osems[b])

    def compute(c, b):
        mrows_v = mrows[b]
        out_v = outs[b]

        @pl.when(c >= 2)
        def _():
            out_desc(c - 2, b).wait()

        def user_body(i, _):
            g = (c * CHUNK_U + i) * D
            u0 = urows_v[pl.ds(g, 16)]
            u1 = urows_v[pl.ds(g + 16, 16)]
            u2 = urows_v[pl.ds(g + 32, 16)]
            u3 = urows_v[pl.ds(g + 48, 16)]
            base_row = i * HIST

            def dot(r):
                return (mrows_v[r, pl.ds(0, 16)] * u0
                        + mrows_v[r, pl.ds(16, 16)] * u1
                        + mrows_v[r, pl.ds(32, 16)] * u2
                        + mrows_v[r, pl.ds(48, 16)] * u3)

            for h in range(0, HIST, 2):
                ra = base_row + h
                pa = dot(ra)
                pb = dot(ra + 1)
                sa = pa + pa.at[bfly[0]].get(mode="promise_in_bounds")
                sb = pb + pb.at[bfly[0]].get(mode="promise_in_bounds")
                s = jnp.where(lo_half, sa, sb)
                for ix in bfly[1:]:
                    s = s + s.at[ix].get(mode="promise_in_bounds")
                plsc.store_scatter(out_v, [out_off + ra], s, mask=out_mask)
            return 0

        lax.fori_loop(0, CHUNK_U, user_body, 0)
        out_desc(c, b).start()

    pltpu.sync_copy(midx_hbm.at[pl.ds(wbase * HIST, U_PER_W * HIST)], midx_v)
    issue(0, 0)
    pltpu.sync_copy(urows_hbm.at[pl.ds(wbase * D, U_PER_W * D)], urows_v)

    def pair_body(gp, _):
        c0 = gp * 2
        issue(c0 + 1, 1)
        wait(0)
        compute(c0, 0)

        @pl.when(c0 + 2 < N_CHUNKS)
        def _():
            issue(c0 + 2, 0)

        wait(1)
        compute(c0 + 1, 1)
        return 0

    lax.fori_loop(0, N_CHUNKS // 2, pair_body, 0)

    out_desc(N_CHUNKS - 2, 0).wait()
    out_desc(N_CHUNKS - 1, 1).wait()


def _user_body(uid_hbm, utab_hbm, out_hbm, uidx_v, acc_v, *scratch):
    stages = scratch[:URING]
    sems = scratch[URING:]
    wid = lax.axis_index("s") * NC + lax.axis_index("c")
    wbase = wid * U_PER_W

    pltpu.sync_copy(uid_hbm.at[pl.ds(wbase, U_PER_W)],
                    uidx_v.at[pl.ds(0, U_PER_W)])

    def fetch(i, j):
        u = uidx_v[pl.ds(i, 16)][0]
        rowg = pl.multiple_of((u >> 3) * 8, 8)
        pltpu.make_async_copy(utab_hbm.at[pl.ds(rowg, 8), :],
                              stages[j], sems[j]).start()

    for j in range(URING):
        fetch(j, j)

    def outer(o, _):
        for j in range(URING):
            i = o * URING + j
            pltpu.make_async_copy(utab_hbm.at[pl.ds(0, 8), :],
                                  stages[j], sems[j]).wait()
            u = uidx_v[pl.ds(i, 16)][0]
            r = u & 7
            for c in range(4):
                acc_v[pl.ds(i * D + c * 16, 16)] = stages[j][r, pl.ds(c * 16, 16)]

            @pl.when(i + URING < U_PER_W)
            def _():
                fetch(i + URING, j)
        return 0

    lax.fori_loop(0, U_PER_W // URING, outer, 0)
    pltpu.sync_copy(acc_v, out_hbm.at[pl.ds(wbase * D, U_PER_W * D)])


@jax.jit
def _run(uid_flat, midx_flat, user_table, movie_table):
    mesh = plsc.VectorSubcoreMesh(core_axis_name="c", subcore_axis_name="s")
    ku = pl.kernel(
        _user_body,
        out_type=jax.ShapeDtypeStruct((B * D,), jnp.float32),
        mesh=mesh,
        scratch_types=[
            pltpu.VMEM((U_PER_W + 16,), jnp.int32),
            pltpu.VMEM((U_PER_W * D,), jnp.float32),
        ] + [pltpu.VMEM((8, D), jnp.float32) for _ in range(URING)]
          + [pltpu.SemaphoreType.DMA for _ in range(URING)],
        compiler_params=pltpu.CompilerParams(needs_layout_passes=False,
                                             use_tc_tiling_on_sc=True),
    )
    urows = ku(uid_flat, user_table)

    k = pl.kernel(
        _sc_body,
        out_type=jax.ShapeDtypeStruct((B * HIST,), jnp.float32),
        mesh=mesh,
        scratch_types=[
            pltpu.VMEM((U_PER_W * HIST,), jnp.int32),
            pltpu.VMEM((U_PER_W * D,), jnp.float32),
            pltpu.VMEM((ROWS, D), jnp.float32),
            pltpu.VMEM((ROWS, D), jnp.float32),
            pltpu.VMEM((ROWS,), jnp.float32),
            pltpu.VMEM((ROWS,), jnp.float32),
            pltpu.SemaphoreType.DMA,
            pltpu.SemaphoreType.DMA,
            pltpu.SemaphoreType.DMA,
            pltpu.SemaphoreType.DMA,
        ],
        compiler_params=pltpu.CompilerParams(needs_layout_passes=False,
                                             use_tc_tiling_on_sc=False),
    )
    return k(urows, midx_flat, movie_table)


def kernel(user_id, movie_title, user_table, movie_table):
    uid_flat = user_id.reshape(B)
    midx_flat = movie_title.reshape(B * HIST)
    out = _run(uid_flat, midx_flat, user_table, movie_table)
    return out.reshape(B, HIST)


# speedup vs baseline: 2.0709x; 1.1962x over previous
import jax
import jax.numpy as jnp
from jax import lax
from jax.experimental import pallas as pl
from jax.experimental.pallas import tpu as pltpu
from jax.experimental.pallas import tpu_sc as plsc

B = 16384
HIST = 50
D = 64

NC = 2
NS = 16
NW = NC * NS

U_PER_W = B // NW
CHUNK_U = 8
N_CHUNKS = U_PER_W // CHUNK_U
ROWS = CHUNK_U * HIST
IDX_SPLIT = (128, 128, 128, 16)
URING = 8


def _sc_body(urows_hbm, midx_hbm, mtab_hbm, out_hbm,
             midx_v, urows_v,
             mrows0_v, mrows1_v, out0_v, out1_v,
             gsem0, gsem1, osem0, osem1):
    mrows = (mrows0_v, mrows1_v)
    outs = (out0_v, out1_v)
    gsems = (gsem0, gsem1)
    osems = (osem0, osem1)
    wid = lax.axis_index("s") * NC + lax.axis_index("c")
    wbase = wid * U_PER_W

    lane = lax.iota(jnp.int32, 16)
    out_mask = (lane == 7) | (lane == 15)
    lo_half = lane < 8
    out_off = jnp.where(lo_half, 0, 1)
    bfly = [lane ^ d for d in (8, 4, 2, 1)]

    def movie_descs(c, b):
        descs = []
        off = 0
        for n in IDX_SPLIT:
            descs.append(pltpu.make_async_copy(
                mtab_hbm.at[midx_v.at[pl.ds(c * ROWS + off, n)]],
                mrows[b].at[pl.ds(off, n)], gsems[b]))
            off += n
        return descs

    def issue(c, b):
        for d in movie_descs(c, b):
            d.start()

    def wait(b):
        for d in movie_descs(0, b):
            d.wait()

    def out_desc(c, b):
        return pltpu.make_async_copy(
            outs[b], out_hbm.at[pl.ds((wbase + c * CHUNK_U) * HIST, ROWS)],
            osems[b])

    def compute(c, b):
        mrows_v = mrows[b]
        out_v = outs[b]

        @pl.when(c >= 2)
        def _():
            out_desc(c - 2, b).wait()

        def user_body(i, _):
            g = (c * CHUNK_U + i) * D
            u0 = urows_v[pl.ds(g, 16)]
            u1 = urows_v[pl.ds(g + 16, 16)]
            u2 = urows_v[pl.ds(g + 32, 16)]
            u3 = urows_v[pl.ds(g + 48, 16)]
            base_row = i * HIST

            def dot(r):
                return (mrows_v[r, pl.ds(0, 16)] * u0
                        + mrows_v[r, pl.ds(16, 16)] * u1
                        + mrows_v[r, pl.ds(32, 16)] * u2
                        + mrows_v[r, pl.ds(48, 16)] * u3)

            for h in range(0, HIST, 2):
                ra = base_row + h
                pa = dot(ra)
                pb = dot(ra + 1)
                sa = pa + pa.at[bfly[0]].get(mode="promise_in_bounds")
                sb = pb + pb.at[bfly[0]].get(mode="promise_in_bounds")
                s = jnp.where(lo_half, sa, sb)
                for ix in bfly[1:]:
                    s = s + s.at[ix].get(mode="promise_in_bounds")
                plsc.store_scatter(out_v, [out_off + ra], s, mask=out_mask)
            return 0

        lax.fori_loop(0, CHUNK_U, user_body, 0)
        out_desc(c, b).start()

    pltpu.sync_copy(midx_hbm.at[pl.ds(wbase * HIST, U_PER_W * HIST)], midx_v)
    issue(0, 0)
    pltpu.sync_copy(urows_hbm.at[pl.ds(wbase * D, U_PER_W * D)], urows_v)

    def pair_body(gp, _):
        c0 = gp * 2
        issue(c0 + 1, 1)
        wait(0)
        compute(c0, 0)

        @pl.when(c0 + 2 < N_CHUNKS)
        def _():
            issue(c0 + 2, 0)

        wait(1)
        compute(c0 + 1, 1)
        return 0

    lax.fori_loop(0, N_CHUNKS // 2, pair_body, 0)

    out_desc(N_CHUNKS - 2, 0).wait()
    out_desc(N_CHUNKS - 1, 1).wait()


def _user_body(uid_hbm, utabT_hbm, out_hbm, uidx_v, acc_v, *scratch):
    stages = scratch[:URING]
    sems = scratch[URING:]
    wid = lax.axis_index("s") * NC + lax.axis_index("c")
    wbase = wid * U_PER_W
    lane = lax.iota(jnp.int32, 16)

    pltpu.sync_copy(uid_hbm.at[pl.ds(wbase, U_PER_W)],
                    uidx_v.at[pl.ds(0, U_PER_W)])

    def fetch(i, j):
        u = uidx_v[pl.ds(i, 16)][0]
        colg = pl.multiple_of((u >> 7) * 128, 128)
        pltpu.make_async_copy(utabT_hbm.at[:, pl.ds(colg, 128)],
                              stages[j], sems[j]).start()

    for j in range(URING):
        fetch(j, j)

    def outer(o, _):
        for j in range(URING):
            i = o * URING + j
            pltpu.make_async_copy(utabT_hbm.at[:, pl.ds(0, 128)],
                                  stages[j], sems[j]).wait()
            u = uidx_v[pl.ds(i, 16)][0]
            col = jnp.full((16,), u & 127, jnp.int32)
            for c in range(4):
                acc_v[pl.ds(i * D + c * 16, 16)] = plsc.load_gather(
                    stages[j], [c * 16 + lane, col])

            @pl.when(i + URING < U_PER_W)
            def _():
                fetch(i + URING, j)
        return 0

    lax.fori_loop(0, U_PER_W // URING, outer, 0)
    pltpu.sync_copy(acc_v, out_hbm.at[pl.ds(wbase * D, U_PER_W * D)])


@jax.jit
def _run(uid_flat, midx_flat, user_table_t, movie_table):
    mesh = plsc.VectorSubcoreMesh(core_axis_name="c", subcore_axis_name="s")
    ku = pl.kernel(
        _user_body,
        out_type=jax.ShapeDtypeStruct((B * D,), jnp.float32),
        mesh=mesh,
        scratch_types=[
            pltpu.VMEM((U_PER_W + 16,), jnp.int32),
            pltpu.VMEM((U_PER_W * D,), jnp.float32),
        ] + [pltpu.VMEM((D, 128), jnp.float32) for _ in range(URING)]
          + [pltpu.SemaphoreType.DMA for _ in range(URING)],
        compiler_params=pltpu.CompilerParams(needs_layout_passes=False,
                                             use_tc_tiling_on_sc=True),
    )
    urows = ku(uid_flat, user_table_t)

    k = pl.kernel(
        _sc_body,
        out_type=jax.ShapeDtypeStruct((B * HIST,), jnp.float32),
        mesh=mesh,
        scratch_types=[
            pltpu.VMEM((U_PER_W * HIST,), jnp.int32),
            pltpu.VMEM((U_PER_W * D,), jnp.float32),
            pltpu.VMEM((ROWS, D), jnp.float32),
            pltpu.VMEM((ROWS, D), jnp.float32),
            pltpu.VMEM((ROWS,), jnp.float32),
            pltpu.VMEM((ROWS,), jnp.float32),
            pltpu.SemaphoreType.DMA,
            pltpu.SemaphoreType.DMA,
            pltpu.SemaphoreType.DMA,
            pltpu.SemaphoreType.DMA,
        ],
        compiler_params=pltpu.CompilerParams(needs_layout_passes=False,
                                             use_tc_tiling_on_sc=False),
    )
    return k(urows, midx_flat, movie_table)


def kernel(user_id, movie_title, user_table, movie_table):
    uid_flat = user_id.reshape(B)
    midx_flat = movie_title.reshape(B * HIST)
    out = _run(uid_flat, midx_flat, user_table.T, movie_table)
    return out.reshape(B, HIST)
